# baseline (speedup 1.0000x reference)
import functools

import jax
import jax.numpy as jnp
import numpy as np
from jax import lax
from jax.experimental import pallas as pl
from jax.experimental.pallas import tpu as pltpu
from jax.experimental.pallas import tpu_sc as plsc


_NCORES = 2
_NSUB = 16
_BLK = 128
_SPMEM_WORDS = 2_080_000


def _pick_depth(NB, slot_words, acc_words):
    for D in (8, 6, 4, 3, 2):
        if NB % D:
            continue
        per_sub = NB * _BLK * 3 + D * slot_words
        if 16 * per_sub + acc_words <= _SPMEM_WORDS and per_sub <= 96_000:
            return D
    return 2


def _weight_rows(fused, sp, src_bufs, dst_buf, w_v, j, C):
    def group(g, carry):
        wv16 = w_v[j, pl.ds(g * 16, 16)]
        for ii in range(16):
            i = g * 16 + ii
            wspl = jnp.full((16,), wv16[ii], jnp.float32)
            for cc in range(C // 16):
                sl = pl.ds(cc * 16, 16)
                slh = pl.ds(cc * 16 + C, 16)
                if fused == 0:
                    v = src_bufs[0][i, sl]
                elif fused == 1:
                    v = sp * (src_bufs[0][i, sl] + src_bufs[0][i, slh])
                else:
                    v = src_bufs[0][i, sl] + sp * (src_bufs[1][i, sl]
                                                   + src_bufs[1][i, slh])
                dst_buf[i, sl] = v * wspl
        return carry

    lax.fori_loop(0, _BLK // 16, group, 0)


@functools.lru_cache(maxsize=None)
def _make_lap(N, C, E, fused, sp):
    NB = E // _BLK // (_NCORES * _NSUB)
    NZ = N // _NSUB
    NG = 1 if fused == 1 else (1 if fused == 0 else 2)
    n_in = (2 if fused == 2 else 1) + 4
    gwide = 2 * C if fused else C
    slot_words = _BLK * C
    slot_words += _BLK * gwide + (_BLK * C if fused == 2 else 0)
    D = _pick_depth(NB, slot_words, N * C)
    mesh = plsc.VectorSubcoreMesh(core_axis_name="c", subcore_axis_name="s")

    gbuf = [pltpu.VMEM((_BLK, gwide), jnp.float32)]
    if fused == 2:
        gbuf = [pltpu.VMEM((_BLK, C), jnp.float32)] + gbuf
    scratch = [
        pltpu.VMEM((NB, _BLK), jnp.int32),
        pltpu.VMEM((NB, _BLK), jnp.int32),
        pltpu.VMEM((NB, _BLK), jnp.float32),
        [gbuf] * D,
        [pltpu.VMEM((_BLK, C), jnp.float32)] * D,
        pltpu.VMEM_SHARED((N, C), jnp.float32),
        [pltpu.SemaphoreType.DMA] * D,
        [pltpu.SemaphoreType.DMA] * D,
    ]

    @functools.partial(
        pl.kernel,
        out_type=jax.ShapeDtypeStruct((N, _NCORES * C), jnp.float32),
        mesh=mesh,
        scratch_types=scratch,
        compiler_params=pltpu.CompilerParams(use_tc_tiling_on_sc=False),
    )
    def lap_k(*args):
        ins = args[:n_in]
        if fused == 2:
            a, pp = ins[0], ins[1]
        elif fused == 1:
            a, pp = None, ins[0]
        else:
            a, pp = ins[0], None
        srcb, dstb, wb, zeros = ins[n_in - 4:]
        rest = list(args[n_in:])
        out = rest.pop(0)
        src_v = rest.pop(0)
        dst_v = rest.pop(0)
        w_v = rest.pop(0)
        gb = rest.pop(0)
        sb = rest.pop(0)
        acc = rest.pop(0)
        gsem = rest.pop(0)
        ssem = rest.pop(0)

        cid = lax.axis_index("c")
        sid = lax.axis_index("s")
        wid = cid * _NSUB + sid
        boff = wid * NB
        pltpu.sync_copy(zeros.at[pl.ds(sid * NZ, NZ)], acc.at[pl.ds(sid * NZ, NZ)])
        pltpu.sync_copy(srcb.at[pl.ds(boff, NB)], src_v)
        pltpu.sync_copy(dstb.at[pl.ds(boff, NB)], dst_v)
        pltpu.sync_copy(wb.at[pl.ds(boff, NB)], w_v)
        plsc.subcore_barrier()

        def gathers(j, b):
            if fused == 0:
                srcs = ((a, 0),)
            elif fused == 1:
                srcs = ((pp, 0),)
            else:
                srcs = ((a, 0), (pp, 1))
            return [pltpu.make_async_copy(arr.at[src_v.at[j]], gb[b][ai],
                                          gsem[b])
                    for arr, ai in srcs]

        def issue(j, b):
            for d in gathers(j, b):
                d.start()

        for b in range(D):
            issue(b, b)

        def substep(j, b):
            for d in gathers(j, b):
                d.wait()
            @pl.when(j >= D)
            def _():
                pltpu.make_async_copy(sb[b], acc.at[dst_v.at[j]],
                                      ssem[b]).wait()
            if fused == 2:
                _weight_rows(fused, sp, (gb[b][0], gb[b][1]), sb[b], w_v, j, C)
            else:
                _weight_rows(fused, sp, (gb[b][0],), sb[b], w_v, j, C)
            jn = jnp.minimum(j + D, NB - 1)
            issue(jn, b)
            pltpu.async_copy(sb[b], acc.at[dst_v.at[j]], ssem[b], add=True)

        def block_body(k, carry):
            for b in range(D):
                substep(D * k + b, b)
            return carry

        lax.fori_loop(0, NB // D, block_body, 0)
        for b in range(D):
            for d in gathers(0, b):
                d.wait()
            pltpu.make_async_copy(sb[b], acc.at[dst_v.at[0]], ssem[b]).wait()
        plsc.subcore_barrier()
        pltpu.sync_copy(acc.at[pl.ds(sid * NZ, NZ)],
                        out.at[pl.ds(sid * NZ, NZ), pl.ds(cid * C, C)])

    return lap_k


def _lap_raw(z, s2d, d2d, w2d, N, C):
    E = w2d.size
    zeros = jnp.zeros((N, C), jnp.float32)
    return _make_lap(N, C, E, 0, 1.0)(z, s2d, d2d, w2d, zeros)


def _lapx(a, P, s2d, d2d, w2d, N, C, sp):
    E = w2d.size
    zeros = jnp.zeros((N, C), jnp.float32)
    if a is None:
        return _make_lap(N, C, E, 1, sp)(P, s2d, d2d, w2d, zeros)
    return _make_lap(N, C, E, 2, sp)(a, P, s2d, d2d, w2d, zeros)


def _cheb_pair(u1, u2, s2d, d2d, w2d, N, C):
    if N * C * 4 > 5_000_000 and C > 16:
        h = C // 2
        qa = _cheb_pair(u1[:, :h], u2[:, :h], s2d, d2d, w2d, N, h)
        qb = _cheb_pair(u1[:, h:], u2[:, h:], s2d, d2d, w2d, N, h)
        return jnp.concatenate([qa, qb], axis=1)
    P = _lap_raw(u2, s2d, d2d, w2d, N, C)
    return _lapx(u1, P, s2d, d2d, w2d, N, C, 2.0)


def _zero_map(r, i):
    return (0,) * r


def _off_map(o, i):
    return (i + o, 0)


def _tc(body, grid, ins, outs):
    args, in_specs = [], []
    for item in ins:
        a, br = item[0], item[1]
        off = item[2] if len(item) > 2 else 0
        args.append(a)
        if br is None:
            in_specs.append(pl.BlockSpec(a.shape, functools.partial(_zero_map, a.ndim)))
        else:
            in_specs.append(pl.BlockSpec((br, a.shape[1]),
                                         functools.partial(_off_map, off)))
    out_specs, out_shapes = [], []
    for shp, br in outs:
        out_shapes.append(jax.ShapeDtypeStruct(shp, jnp.float32))
        if br is None:
            out_specs.append(pl.BlockSpec(shp, functools.partial(_zero_map, len(shp))))
        else:
            out_specs.append(pl.BlockSpec((br, shp[1]),
                                          functools.partial(_off_map, 0)))
    return pl.pallas_call(
        body,
        grid=(grid,),
        in_specs=in_specs,
        out_specs=out_specs,
        out_shape=out_shapes,
    )(*args)


def _dot(a, b):
    return jnp.dot(a, b, preferred_element_type=jnp.float32)


def _relu(v):
    return jnp.maximum(v, 0.0)


def _pre3_body(x_ref, t1_ref, t2_ref, t3_ref, a1_ref, a2_ref, a3_ref):
    xb = x_ref[...]
    a1_ref[...] = _dot(xb, t1_ref[...])
    a2_ref[...] = _dot(xb, t2_ref[...])
    a3_ref[...] = _dot(xb, t3_ref[...])


def _c1post_body(a3_ref, q_ref, b_ref, t_ref, s_ref, s2_ref):
    q = q_ref[...]
    qs = jnp.concatenate([q[:, 0:16] + q[:, 16:32],
                          q[:, 32:48] + q[:, 48:64]], axis=1)
    tb = a3_ref[...] + qs + b_ref[...]
    t_ref[...] = tb

    @pl.when(pl.program_id(0) == 0)
    def _():
        s_ref[...] = jnp.zeros_like(s_ref)
        s2_ref[...] = jnp.zeros_like(s2_ref)

    s_ref[...] += jnp.sum(tb, axis=0, keepdims=True)
    s2_ref[...] += jnp.sum(tb * tb, axis=0, keepdims=True)


def _make_gnpre_body(eps=1e-5):
    def body(t_ref, s_ref, s2_ref, pg_ref, gm_ref, bt_ref,
             wa_ref, ba_ref, t1_ref, t2_ref, t3_ref,
             g_ref, u1_ref, u2_ref, u3_ref):
        m_ch = _dot(s_ref[...], pg_ref[...])
        ex2_ch = _dot(s2_ref[...], pg_ref[...])
        var_ch = ex2_ch - m_ch * m_ch
        rstd = lax.rsqrt(var_ch + eps)
        gb = _relu((t_ref[...] - m_ch) * rstd * gm_ref[...] + bt_ref[...])
        g_ref[...] = gb
        h = _relu(_dot(gb, wa_ref[...]) + ba_ref[...])
        u1_ref[...] = _dot(h, t1_ref[...])
        u2_ref[...] = _dot(h, t2_ref[...])
        u3_ref[...] = _dot(h, t3_ref[...])
    return body


def _make_postpre_body(h, cp):
    def body(u3_ref, q_ref, bc_ref, wb_ref, bb_ref, g_ref, wr_ref,
             br_ref, wa_ref, ba_ref, t1_ref, t2_ref, t3_ref,
             y_ref, v1_ref, v2_ref, v3_ref):
        q = q_ref[...]
        cheb = _relu(u3_ref[...] + q[:, :h] + q[:, cp:cp + h] + bc_ref[...])
        y = _relu(_dot(cheb, wb_ref[...]) + bb_ref[...]
                  + _dot(g_ref[...], wr_ref[...]) + br_ref[...])
        y_ref[...] = y
        h2 = _relu(_dot(y, wa_ref[...]) + ba_ref[...])
        v1_ref[...] = _dot(h2, t1_ref[...])
        v2_ref[...] = _dot(h2, t2_ref[...])
        v3_ref[...] = _dot(h2, t3_ref[...])
    return body


def _make_post_body(h, cp):
    def body(u3_ref, q_ref, bc_ref, wb_ref, bb_ref, g_ref, wr_ref,
             br_ref, y_ref):
        q = q_ref[...]
        cheb = _relu(u3_ref[...] + q[:, :h] + q[:, cp:cp + h] + bc_ref[...])
        y_ref[...] = _relu(_dot(cheb, wb_ref[...]) + bb_ref[...]
                           + _dot(g_ref[...], wr_ref[...]) + br_ref[...])
    return body


def _make_poolpre_body(C):
    def body(yr_ref, wa_ref, ba_ref, t1_ref, t2_ref, t3_ref,
             p_ref, u1_ref, u2_ref, u3_ref):
        yr = yr_ref[...]
        pb = 0.25 * (yr[:, :C] + yr[:, C:2 * C] + yr[:, 2 * C:3 * C] + yr[:, 3 * C:])
        p_ref[...] = pb
        h = _relu(_dot(pb, wa_ref[...]) + ba_ref[...])
        u1_ref[...] = _dot(h, t1_ref[...])
        u2_ref[...] = _dot(h, t2_ref[...])
        u3_ref[...] = _dot(h, t3_ref[...])
    return body


def _conv2_body(y_ref, p_ref, q_ref, t3_ref, t1a_ref, t1b_ref, t2a_ref,
                t2b_ref, b_ref, o_ref):
    p = p_ref[...]
    q = q_ref[...]
    o_ref[...] = (_dot(y_ref[...], t3_ref[...])
                  + _dot(p[:, 0:64] + p[:, 64:128], t1a_ref[...])
                  + _dot(p[:, 128:192] + p[:, 192:256], t1b_ref[...])
                  + _dot(2.0 * (q[:, 0:64] + q[:, 64:128]), t2a_ref[...])
                  + _dot(2.0 * (q[:, 128:192] + q[:, 192:256]), t2b_ref[...])
                  + b_ref[...])


def _make_poolfin_body(C):
    def body(yr_ref, o_ref):
        yr = yr_ref[...]
        o_ref[...] = 0.25 * (yr[:, :C] + yr[:, C:2 * C]
                             + yr[:, 2 * C:3 * C] + yr[:, 3 * C:])
    return body


def _row2(v):
    return v.reshape(1, -1)


def _theta_pre(p, hp):
    th = p["theta"]
    h = th.shape[1]
    pad = hp - h
    t1 = jnp.pad(th[1], ((0, 0), (0, pad))) if pad else th[1]
    t2 = jnp.pad(th[2], ((0, 0), (0, pad))) if pad else th[2]
    return t1, t2, th[0] - th[2]


def kernel(x, edge_index0, weight0, edge_index1, weight1, edge_index2, weight2,
           params):
    p = params
    z = x[0]
    N0 = z.shape[0]
    N1, N2, N3 = N0 // 4, N0 // 16, N0 // 64
    T0, T1, T2 = 2048, 2048, 1024
    G0, G1, G2 = N0 // T0, N1 // T1, N2 // T2

    s0 = edge_index0[0].reshape(-1, _BLK)
    d0 = edge_index0[1].reshape(-1, _BLK)
    w0 = weight0.reshape(-1, _BLK)
    s1 = edge_index1[0].reshape(-1, _BLK)
    d1 = edge_index1[1].reshape(-1, _BLK)
    w1 = weight1.reshape(-1, _BLK)
    s2 = edge_index2[0].reshape(-1, _BLK)
    d2 = edge_index2[1].reshape(-1, _BLK)
    w2 = weight2.reshape(-1, _BLK)

    th = p["conv1_theta"]
    a1, a2, a3 = _tc(
        _pre3_body, G0,
        [(z, T0), (th[1], None), (th[2], None), (th[0] - th[2], None)],
        [((N0, 32), T0)] * 3)
    Q = _cheb_pair(a1, a2, s0, d0, w0, N0, 32)
    t, ssum, ssq = _tc(
        _c1post_body, G0,
        [(a3, T0), (Q, T0), (_row2(p["conv1_bias"]), None)],
        [((N0, 32), T0), ((1, 32), None), ((1, 32), None)])

    pg = jnp.asarray(np.kron(np.eye(8, dtype=np.float32),
                             np.ones((4, 4), np.float32)) / float(N0 * 4))

    b = p["l1b1"]
    t1p, t2p, t3p = _theta_pre(b, 16)
    g, u1, u2, u3 = _tc(
        _make_gnpre_body(), G0,
        [(t, T0), (ssum, None), (ssq, None), (pg, None),
         (_row2(p["gamma"]), None), (_row2(p["beta"]), None),
         (b["Wa"], None), (_row2(b["ba"]), None),
         (t1p, None), (t2p, None), (t3p, None)],
        [((N0, 32), T0), ((N0, 16), T0), ((N0, 16), T0), ((N0, 8), T0)])

    Q = _cheb_pair(u1, u2, s0, d0, w0, N0, 16)
    b2 = p["l1b2"]
    t1p, t2p, t3p = _theta_pre(b2, 16)
    y1, u1, u2, u3 = _tc(
        _make_postpre_body(8, 16), G0,
        [(u3, T0), (Q, T0), (_row2(b["bc"]), None),
         (b["Wb"], None), (_row2(b["bb"]), None), (g, T0), (b["Wr"], None),
         (_row2(b["br"]), None), (b2["Wa"], None), (_row2(b2["ba"]), None),
         (t1p, None), (t2p, None), (t3p, None)],
        [((N0, 32), T0), ((N0, 16), T0), ((N0, 16), T0), ((N0, 8), T0)])

    Q = _cheb_pair(u1, u2, s0, d0, w0, N0, 16)
    y2 = _tc(
        _make_post_body(8, 16), G0,
        [(u3, T0), (Q, T0), (_row2(b2["bc"]), None),
         (b2["Wb"], None), (_row2(b2["bb"]), None), (y1, T0), (b2["Wr"], None),
         (_row2(b2["br"]), None)],
        [((N0, 32), T0)])[0]

    b = p["l2b1"]
    t1p, t2p, t3p = _theta_pre(b, 16)
    yr = y2.reshape(N1, 4 * 32)
    g, u1, u2, u3 = _tc(
        _make_poolpre_body(32), G1,
        [(yr, T1), (b["Wa"], None), (_row2(b["ba"]), None),
         (t1p, None), (t2p, None), (t3p, None)],
        [((N1, 32), T1), ((N1, 16), T1), ((N1, 16), T1), ((N1, 16), T1)])

    Q = _cheb_pair(u1, u2, s1, d1, w1, N1, 16)
    b2 = p["l2b2"]
    t1p, t2p, t3p = _theta_pre(b2, 16)
    y1, u1, u2, u3 = _tc(
        _make_postpre_body(16, 16), G1,
        [(u3, T1), (Q, T1), (_row2(b["bc"]), None),
         (b["Wb"], None), (_row2(b["bb"]), None), (g, T1), (b["Wr"], None),
         (_row2(b["br"]), None), (b2["Wa"], None), (_row2(b2["ba"]), None),
         (t1p, None), (t2p, None), (t3p, None)],
        [((N1, 64), T1), ((N1, 16), T1), ((N1, 16), T1), ((N1, 16), T1)])

    Q = _cheb_pair(u1, u2, s1, d1, w1, N1, 16)
    y2 = _tc(
        _make_post_body(16, 16), G1,
        [(u3, T1), (Q, T1), (_row2(b2["bc"]), None),
         (b2["Wb"], None), (_row2(b2["bb"]), None), (y1, T1), (b2["Wr"], None),
         (_row2(b2["br"]), None)],
        [((N1, 64), T1)])[0]

    b = p["l3b1"]
    t1p, t2p, t3p = _theta_pre(b, 32)
    yr = y2.reshape(N2, 4 * 64)
    g, u1, u2, u3 = _tc(
        _make_poolpre_body(64), G2,
        [(yr, T2), (b["Wa"], None), (_row2(b["ba"]), None),
         (t1p, None), (t2p, None), (t3p, None)],
        [((N2, 64), T2), ((N2, 32), T2), ((N2, 32), T2), ((N2, 32), T2)])

    Q = _cheb_pair(u1, u2, s2, d2, w2, N2, 32)
    b2 = p["l3b2"]
    t1p, t2p, t3p = _theta_pre(b2, 32)
    y1, u1, u2, u3 = _tc(
        _make_postpre_body(32, 32), G2,
        [(u3, T2), (Q, T2), (_row2(b["bc"]), None),
         (b["Wb"], None), (_row2(b["bb"]), None), (g, T2), (b["Wr"], None),
         (_row2(b["br"]), None), (b2["Wa"], None), (_row2(b2["ba"]), None),
         (t1p, None), (t2p, None), (t3p, None)],
        [((N2, 128), T2), ((N2, 32), T2), ((N2, 32), T2), ((N2, 32), T2)])

    Q = _cheb_pair(u1, u2, s2, d2, w2, N2, 32)
    y2 = _tc(
        _make_post_body(32, 32), G2,
        [(u3, T2), (Q, T2), (_row2(b2["bc"]), None),
         (b2["Wb"], None), (_row2(b2["bb"]), None), (y1, T2), (b2["Wr"], None),
         (_row2(b2["br"]), None)],
        [((N2, 128), T2)])[0]

    Ps, Qs = [], []
    for lo in (0, 64):
        zh = y2[:, lo:lo + 64]
        Ph = _lap_raw(zh, s2, d2, w2, N2, 64)
        Qs.append(_lapx(None, Ph, s2, d2, w2, N2, 64, 1.0))
        Ps.append(Ph)
    P = jnp.concatenate(Ps, axis=1)
    Q = jnp.concatenate(Qs, axis=1)
    th2 = p["conv2_theta"]
    o = _tc(
        _conv2_body, G2,
        [(y2, T2), (P, T2), (Q, T2),
         (th2[0] - th2[2], None), (th2[1][:64], None), (th2[1][64:], None),
         (th2[2][:64], None), (th2[2][64:], None),
         (_row2(p["conv2_bias"]), None)],
        [((N2, 256), T2)])[0]

    out = _tc(_make_poolfin_body(256), 1,
              [(o.reshape(N3, 4 * 256), None)],
              [((N3, 256), None)])[0]
    return out[None]


# speedup vs baseline: 1.2686x; 1.2686x over previous
import functools

import jax
import jax.numpy as jnp
import numpy as np
from jax import lax
from jax.experimental import pallas as pl
from jax.experimental.pallas import tpu as pltpu
from jax.experimental.pallas import tpu_sc as plsc


_NCORES = 2
_NSUB = 16
_BLK = 128
_SPMEM_WORDS = 2_080_000


def _pick_depth(NB, C, n_gather, n_idx, acc_words):
    for D in (8, 6, 4, 3, 2):
        if NB % D:
            continue
        per_sub = NB * _BLK * n_idx + D * _BLK * C * (n_gather + 1)
        if 16 * per_sub + acc_words <= _SPMEM_WORDS and per_sub <= 96_000:
            return D
    return 2


def _weight_rows(src_bufs, dst_buf, w_v, j, C, sp):
    def group(g, carry):
        wv16 = w_v[j, pl.ds(g * 16, 16)]
        for ii in range(16):
            i = g * 16 + ii
            wspl = jnp.full((16,), wv16[ii], jnp.float32)
            for cc in range(C // 16):
                sl = pl.ds(cc * 16, 16)
                if len(src_bufs) == 1:
                    v = src_bufs[0][i, sl]
                elif len(src_bufs) == 2:
                    v = sp * (src_bufs[0][i, sl] + src_bufs[1][i, sl])
                else:
                    v = src_bufs[0][i, sl] + sp * (src_bufs[1][i, sl]
                                                   + src_bufs[2][i, sl])
                dst_buf[i, sl] = v * wspl
        return carry

    lax.fori_loop(0, _BLK // 16, group, 0)


@functools.lru_cache(maxsize=None)
def _make_lap(N, C, E, fused, sp):
    NB = E // _BLK // (_NCORES * _NSUB)
    NZ = N // _NSUB
    NG = 1 if fused == 0 else (2 if fused == 1 else 3)
    n_in = (2 if fused == 2 else 1) + 4
    n_idx = 3 + (1 if fused else 0)
    D = _pick_depth(NB, C, NG, n_idx, N * C)
    mesh = plsc.VectorSubcoreMesh(core_axis_name="c", subcore_axis_name="s")

    scratch = [
        pltpu.VMEM((NB, _BLK), jnp.int32),
        pltpu.VMEM((NB, _BLK), jnp.int32),
        pltpu.VMEM((NB, _BLK), jnp.float32),
        [[pltpu.VMEM((_BLK, C), jnp.float32)] * NG] * D,
        [pltpu.VMEM((_BLK, C), jnp.float32)] * D,
        pltpu.VMEM_SHARED((N, C), jnp.float32),
        [pltpu.SemaphoreType.DMA] * D,
        [pltpu.SemaphoreType.DMA] * D,
    ]
    if fused:
        scratch.insert(1, pltpu.VMEM((NB, _BLK), jnp.int32))

    @functools.partial(
        pl.kernel,
        out_type=jax.ShapeDtypeStruct((_NCORES * N, C), jnp.float32),
        mesh=mesh,
        scratch_types=scratch,
        compiler_params=pltpu.CompilerParams(use_tc_tiling_on_sc=False),
    )
    def lap_k(*args):
        ins = args[:n_in]
        if fused == 2:
            a, pp = ins[0], ins[1]
        elif fused == 1:
            a, pp = None, ins[0]
        else:
            a, pp = ins[0], None
        srcb, dstb, wb, zeros = ins[n_in - 4:]
        rest = list(args[n_in:])
        out = rest.pop(0)
        src_v = rest.pop(0)
        srcN_v = rest.pop(0) if fused else None
        dst_v = rest.pop(0)
        w_v = rest.pop(0)
        gb = rest.pop(0)
        sb = rest.pop(0)
        acc = rest.pop(0)
        gsem = rest.pop(0)
        ssem = rest.pop(0)

        cid = lax.axis_index("c")
        sid = lax.axis_index("s")
        wid = cid * _NSUB + sid
        boff = wid * NB
        pltpu.sync_copy(zeros.at[pl.ds(sid * NZ, NZ)], acc.at[pl.ds(sid * NZ, NZ)])
        pltpu.sync_copy(srcb.at[pl.ds(boff, NB)], src_v)
        pltpu.sync_copy(dstb.at[pl.ds(boff, NB)], dst_v)
        pltpu.sync_copy(wb.at[pl.ds(boff, NB)], w_v)
        if fused:
            def shift_row(r, carry):
                for g in range(_BLK // 16):
                    srcN_v[r, pl.ds(g * 16, 16)] = (
                        src_v[r, pl.ds(g * 16, 16)] + N)
                return carry
            lax.fori_loop(0, NB, shift_row, 0)
        plsc.subcore_barrier()

        if fused == 0:
            sources = ((a, src_v),)
        elif fused == 1:
            sources = ((pp, src_v), (pp, srcN_v))
        else:
            sources = ((a, src_v), (pp, src_v), (pp, srcN_v))
        NS = {1: 4, 2: 2, 3: 2}[NG]
        RS = _BLK // NS

        def gathers(j, b):
            ds = []
            for ai, (arr, idxv) in enumerate(sources):
                for k in range(NS):
                    ds.append(pltpu.make_async_copy(
                        arr.at[idxv.at[j, pl.ds(k * RS, RS)]],
                        gb[b][ai].at[pl.ds(k * RS, RS)],
                        gsem[b]))
            return ds

        def issue(j, b):
            for d in gathers(j, b):
                d.start()

        for b in range(D):
            issue(b, b)

        def substep(j, b):
            for d in gathers(j, b):
                d.wait()
            @pl.when(j >= D)
            def _():
                pltpu.make_async_copy(sb[b], acc.at[dst_v.at[j]],
                                      ssem[b]).wait()
            _weight_rows(gb[b], sb[b], w_v, j, C, sp)
            jn = jnp.minimum(j + D, NB - 1)
            issue(jn, b)
            pltpu.async_copy(sb[b], acc.at[dst_v.at[j]], ssem[b], add=True)

        def block_body(k, carry):
            for b in range(D):
                substep(D * k + b, b)
            return carry

        lax.fori_loop(0, NB // D, block_body, 0)
        for b in range(D):
            for d in gathers(0, b):
                d.wait()
            pltpu.make_async_copy(sb[b], acc.at[dst_v.at[0]], ssem[b]).wait()
        plsc.subcore_barrier()
        pltpu.sync_copy(acc.at[pl.ds(sid * NZ, NZ)],
                        out.at[pl.ds(cid * N + sid * NZ, NZ)])

    return lap_k


def _lap_raw(z, s2d, d2d, w2d, N, C):
    E = w2d.size
    zeros = jnp.zeros((N, C), jnp.float32)
    return _make_lap(N, C, E, 0, 1.0)(z, s2d, d2d, w2d, zeros)


def _lapx(a, P, s2d, d2d, w2d, N, C, sp):
    E = w2d.size
    zeros = jnp.zeros((N, C), jnp.float32)
    if a is None:
        return _make_lap(N, C, E, 1, sp)(P, s2d, d2d, w2d, zeros)
    return _make_lap(N, C, E, 2, sp)(a, P, s2d, d2d, w2d, zeros)


def _cheb_pair(u1, u2, s2d, d2d, w2d, N, C):
    if N * C * 4 > 5_000_000 and C > 16:
        h = C // 2
        qa = _cheb_pair(u1[:, :h], u2[:, :h], s2d, d2d, w2d, N, h)
        qb = _cheb_pair(u1[:, h:], u2[:, h:], s2d, d2d, w2d, N, h)
        return jnp.concatenate([qa, qb], axis=1)
    P = _lap_raw(u2, s2d, d2d, w2d, N, C)
    return _lapx(u1, P, s2d, d2d, w2d, N, C, 2.0)


def _zero_map(r, i):
    return (0,) * r


def _off_map(o, i):
    return (i + o, 0)


def _tc(body, grid, ins, outs):
    args, in_specs = [], []
    for item in ins:
        a, br = item[0], item[1]
        off = item[2] if len(item) > 2 else 0
        args.append(a)
        if br is None:
            in_specs.append(pl.BlockSpec(a.shape, functools.partial(_zero_map, a.ndim)))
        else:
            in_specs.append(pl.BlockSpec((br, a.shape[1]),
                                         functools.partial(_off_map, off)))
    out_specs, out_shapes = [], []
    for shp, br in outs:
        out_shapes.append(jax.ShapeDtypeStruct(shp, jnp.float32))
        if br is None:
            out_specs.append(pl.BlockSpec(shp, functools.partial(_zero_map, len(shp))))
        else:
            out_specs.append(pl.BlockSpec((br, shp[1]),
                                          functools.partial(_off_map, 0)))
    return pl.pallas_call(
        body,
        grid=(grid,),
        in_specs=in_specs,
        out_specs=out_specs,
        out_shape=out_shapes,
    )(*args)


def _dot(a, b):
    return jnp.dot(a, b, preferred_element_type=jnp.float32)


def _relu(v):
    return jnp.maximum(v, 0.0)


def _pre3_body(x_ref, t1_ref, t2_ref, t3_ref, a1_ref, a2_ref, a3_ref):
    xb = x_ref[...]
    a1_ref[...] = _dot(xb, t1_ref[...])
    a2_ref[...] = _dot(xb, t2_ref[...])
    a3_ref[...] = _dot(xb, t3_ref[...])


def _c1post_body(a3_ref, q0_ref, q1_ref, b_ref, t_ref, s_ref, s2_ref):
    tb = a3_ref[...] + q0_ref[...] + q1_ref[...] + b_ref[...]
    t_ref[...] = tb

    @pl.when(pl.program_id(0) == 0)
    def _():
        s_ref[...] = jnp.zeros_like(s_ref)
        s2_ref[...] = jnp.zeros_like(s2_ref)

    s_ref[...] += jnp.sum(tb, axis=0, keepdims=True)
    s2_ref[...] += jnp.sum(tb * tb, axis=0, keepdims=True)


def _make_gnpre_body(eps=1e-5):
    def body(t_ref, s_ref, s2_ref, pg_ref, gm_ref, bt_ref,
             wa_ref, ba_ref, t1_ref, t2_ref, t3_ref,
             g_ref, u1_ref, u2_ref, u3_ref):
        m_ch = _dot(s_ref[...], pg_ref[...])
        ex2_ch = _dot(s2_ref[...], pg_ref[...])
        var_ch = ex2_ch - m_ch * m_ch
        rstd = lax.rsqrt(var_ch + eps)
        gb = _relu((t_ref[...] - m_ch) * rstd * gm_ref[...] + bt_ref[...])
        g_ref[...] = gb
        h = _relu(_dot(gb, wa_ref[...]) + ba_ref[...])
        u1_ref[...] = _dot(h, t1_ref[...])
        u2_ref[...] = _dot(h, t2_ref[...])
        u3_ref[...] = _dot(h, t3_ref[...])
    return body


def _make_postpre_body(h):
    def body(u3_ref, q0_ref, q1_ref, bc_ref, wb_ref, bb_ref, g_ref, wr_ref,
             br_ref, wa_ref, ba_ref, t1_ref, t2_ref, t3_ref,
             y_ref, v1_ref, v2_ref, v3_ref):
        cheb = _relu(u3_ref[...] + q0_ref[...][:, :h] + q1_ref[...][:, :h]
                     + bc_ref[...])
        y = _relu(_dot(cheb, wb_ref[...]) + bb_ref[...]
                  + _dot(g_ref[...], wr_ref[...]) + br_ref[...])
        y_ref[...] = y
        h2 = _relu(_dot(y, wa_ref[...]) + ba_ref[...])
        v1_ref[...] = _dot(h2, t1_ref[...])
        v2_ref[...] = _dot(h2, t2_ref[...])
        v3_ref[...] = _dot(h2, t3_ref[...])
    return body


def _make_post_body(h):
    def body(u3_ref, q0_ref, q1_ref, bc_ref, wb_ref, bb_ref, g_ref, wr_ref,
             br_ref, y_ref):
        cheb = _relu(u3_ref[...] + q0_ref[...][:, :h] + q1_ref[...][:, :h]
                     + bc_ref[...])
        y_ref[...] = _relu(_dot(cheb, wb_ref[...]) + bb_ref[...]
                           + _dot(g_ref[...], wr_ref[...]) + br_ref[...])
    return body


def _make_poolpre_body(C):
    def body(yr_ref, wa_ref, ba_ref, t1_ref, t2_ref, t3_ref,
             p_ref, u1_ref, u2_ref, u3_ref):
        yr = yr_ref[...]
        pb = 0.25 * (yr[:, :C] + yr[:, C:2 * C] + yr[:, 2 * C:3 * C] + yr[:, 3 * C:])
        p_ref[...] = pb
        h = _relu(_dot(pb, wa_ref[...]) + ba_ref[...])
        u1_ref[...] = _dot(h, t1_ref[...])
        u2_ref[...] = _dot(h, t2_ref[...])
        u3_ref[...] = _dot(h, t3_ref[...])
    return body


def _conv2_body(y_ref, p0_ref, p1_ref, q0_ref, q1_ref, t3_ref, t1_ref, t2_ref,
                b_ref, o_ref):
    o_ref[...] = (_dot(y_ref[...], t3_ref[...])
                  + _dot(p0_ref[...] + p1_ref[...], t1_ref[...])
                  + _dot(2.0 * (q0_ref[...] + q1_ref[...]), t2_ref[...])
                  + b_ref[...])


def _make_poolfin_body(C):
    def body(yr_ref, o_ref):
        yr = yr_ref[...]
        o_ref[...] = 0.25 * (yr[:, :C] + yr[:, C:2 * C]
                             + yr[:, 2 * C:3 * C] + yr[:, 3 * C:])
    return body


def _row2(v):
    return v.reshape(1, -1)


def _theta_pre(p, hp):
    th = p["theta"]
    h = th.shape[1]
    pad = hp - h
    t1 = jnp.pad(th[1], ((0, 0), (0, pad))) if pad else th[1]
    t2 = jnp.pad(th[2], ((0, 0), (0, pad))) if pad else th[2]
    return t1, t2, th[0] - th[2]


def kernel(x, edge_index0, weight0, edge_index1, weight1, edge_index2, weight2,
           params):
    p = params
    z = x[0]
    N0 = z.shape[0]
    N1, N2, N3 = N0 // 4, N0 // 16, N0 // 64
    T0, T1, T2 = 2048, 2048, 1024
    G0, G1, G2 = N0 // T0, N1 // T1, N2 // T2
    O0, O1, O2 = N0 // T0, N1 // T1, N2 // T2

    s0 = edge_index0[0].reshape(-1, _BLK)
    d0 = edge_index0[1].reshape(-1, _BLK)
    w0 = weight0.reshape(-1, _BLK)
    s1 = edge_index1[0].reshape(-1, _BLK)
    d1 = edge_index1[1].reshape(-1, _BLK)
    w1 = weight1.reshape(-1, _BLK)
    s2 = edge_index2[0].reshape(-1, _BLK)
    d2 = edge_index2[1].reshape(-1, _BLK)
    w2 = weight2.reshape(-1, _BLK)

    th = p["conv1_theta"]
    a1, a2, a3 = _tc(
        _pre3_body, G0,
        [(z, T0), (th[1], None), (th[2], None), (th[0] - th[2], None)],
        [((N0, 32), T0)] * 3)
    Q = _cheb_pair(a1, a2, s0, d0, w0, N0, 32)
    t, ssum, ssq = _tc(
        _c1post_body, G0,
        [(a3, T0), (Q, T0, 0), (Q, T0, O0), (_row2(p["conv1_bias"]), None)],
        [((N0, 32), T0), ((1, 32), None), ((1, 32), None)])

    pg = jnp.asarray(np.kron(np.eye(8, dtype=np.float32),
                             np.ones((4, 4), np.float32)) / float(N0 * 4))

    b = p["l1b1"]
    t1p, t2p, t3p = _theta_pre(b, 16)
    g, u1, u2, u3 = _tc(
        _make_gnpre_body(), G0,
        [(t, T0), (ssum, None), (ssq, None), (pg, None),
         (_row2(p["gamma"]), None), (_row2(p["beta"]), None),
         (b["Wa"], None), (_row2(b["ba"]), None),
         (t1p, None), (t2p, None), (t3p, None)],
        [((N0, 32), T0), ((N0, 16), T0), ((N0, 16), T0), ((N0, 8), T0)])

    Q = _cheb_pair(u1, u2, s0, d0, w0, N0, 16)
    b2 = p["l1b2"]
    t1p, t2p, t3p = _theta_pre(b2, 16)
    y1, u1, u2, u3 = _tc(
        _make_postpre_body(8), G0,
        [(u3, T0), (Q, T0, 0), (Q, T0, O0), (_row2(b["bc"]), None),
         (b["Wb"], None), (_row2(b["bb"]), None), (g, T0), (b["Wr"], None),
         (_row2(b["br"]), None), (b2["Wa"], None), (_row2(b2["ba"]), None),
         (t1p, None), (t2p, None), (t3p, None)],
        [((N0, 32), T0), ((N0, 16), T0), ((N0, 16), T0), ((N0, 8), T0)])

    Q = _cheb_pair(u1, u2, s0, d0, w0, N0, 16)
    y2 = _tc(
        _make_post_body(8), G0,
        [(u3, T0), (Q, T0, 0), (Q, T0, O0), (_row2(b2["bc"]), None),
         (b2["Wb"], None), (_row2(b2["bb"]), None), (y1, T0), (b2["Wr"], None),
         (_row2(b2["br"]), None)],
        [((N0, 32), T0)])[0]

    b = p["l2b1"]
    t1p, t2p, t3p = _theta_pre(b, 16)
    yr = y2.reshape(N1, 4 * 32)
    g, u1, u2, u3 = _tc(
        _make_poolpre_body(32), G1,
        [(yr, T1), (b["Wa"], None), (_row2(b["ba"]), None),
         (t1p, None), (t2p, None), (t3p, None)],
        [((N1, 32), T1), ((N1, 16), T1), ((N1, 16), T1), ((N1, 16), T1)])

    Q = _cheb_pair(u1, u2, s1, d1, w1, N1, 16)
    b2 = p["l2b2"]
    t1p, t2p, t3p = _theta_pre(b2, 16)
    y1, u1, u2, u3 = _tc(
        _make_postpre_body(16), G1,
        [(u3, T1), (Q, T1, 0), (Q, T1, O1), (_row2(b["bc"]), None),
         (b["Wb"], None), (_row2(b["bb"]), None), (g, T1), (b["Wr"], None),
         (_row2(b["br"]), None), (b2["Wa"], None), (_row2(b2["ba"]), None),
         (t1p, None), (t2p, None), (t3p, None)],
        [((N1, 64), T1), ((N1, 16), T1), ((N1, 16), T1), ((N1, 16), T1)])

    Q = _cheb_pair(u1, u2, s1, d1, w1, N1, 16)
    y2 = _tc(
        _make_post_body(16), G1,
        [(u3, T1), (Q, T1, 0), (Q, T1, O1), (_row2(b2["bc"]), None),
         (b2["Wb"], None), (_row2(b2["bb"]), None), (y1, T1), (b2["Wr"], None),
         (_row2(b2["br"]), None)],
        [((N1, 64), T1)])[0]

    b = p["l3b1"]
    t1p, t2p, t3p = _theta_pre(b, 32)
    yr = y2.reshape(N2, 4 * 64)
    g, u1, u2, u3 = _tc(
        _make_poolpre_body(64), G2,
        [(yr, T2), (b["Wa"], None), (_row2(b["ba"]), None),
         (t1p, None), (t2p, None), (t3p, None)],
        [((N2, 64), T2), ((N2, 32), T2), ((N2, 32), T2), ((N2, 32), T2)])

    Q = _cheb_pair(u1, u2, s2, d2, w2, N2, 32)
    b2 = p["l3b2"]
    t1p, t2p, t3p = _theta_pre(b2, 32)
    y1, u1, u2, u3 = _tc(
        _make_postpre_body(32), G2,
        [(u3, T2), (Q, T2, 0), (Q, T2, O2), (_row2(b["bc"]), None),
         (b["Wb"], None), (_row2(b["bb"]), None), (g, T2), (b["Wr"], None),
         (_row2(b["br"]), None), (b2["Wa"], None), (_row2(b2["ba"]), None),
         (t1p, None), (t2p, None), (t3p, None)],
        [((N2, 128), T2), ((N2, 32), T2), ((N2, 32), T2), ((N2, 32), T2)])

    Q = _cheb_pair(u1, u2, s2, d2, w2, N2, 32)
    y2 = _tc(
        _make_post_body(32), G2,
        [(u3, T2), (Q, T2, 0), (Q, T2, O2), (_row2(b2["bc"]), None),
         (b2["Wb"], None), (_row2(b2["bb"]), None), (y1, T2), (b2["Wr"], None),
         (_row2(b2["br"]), None)],
        [((N2, 128), T2)])[0]

    Ps, Qs = [], []
    for lo in (0, 64):
        zh = y2[:, lo:lo + 64]
        Ph = _lap_raw(zh, s2, d2, w2, N2, 64)
        Qs.append(_lapx(None, Ph, s2, d2, w2, N2, 64, 1.0))
        Ps.append(Ph)
    P = jnp.concatenate(Ps, axis=1)
    Q = jnp.concatenate(Qs, axis=1)
    th2 = p["conv2_theta"]
    o = _tc(
        _conv2_body, G2,
        [(y2, T2), (P, T2, 0), (P, T2, O2), (Q, T2, 0), (Q, T2, O2),
         (th2[0] - th2[2], None), (th2[1], None), (th2[2], None),
         (_row2(p["conv2_bias"]), None)],
        [((N2, 256), T2)])[0]

    out = _tc(_make_poolfin_body(256), 1,
              [(o.reshape(N3, 4 * 256), None)],
              [((N3, 256), None)])[0]
    return out[None]
